# baseline (speedup 1.0000x reference)
import functools

import jax
import jax.numpy as jnp
from jax import lax
from jax.experimental import pallas as pl
from jax.experimental.pallas import tpu as pltpu
from jax.experimental.pallas import tpu_sc as plsc

_VOCAB = 1000000
_EMB = 16
_BATCH = 4096
_SEQ = 200
_B = _BATCH * _SEQ
_NC = 2
_NS = 16
_NW = _NC * _NS
_BPW = _B // _NW
_CHUNK = 128
_NG = _BPW // _CHUNK


def _emb_body(idx_hbm, table_hbm, out_hbm, idx_v, row_v, gsem):
    wid = lax.axis_index("s") * _NC + lax.axis_index("c")
    pltpu.sync_copy(idx_hbm.at[wid], idx_v)

    def body(j, carry):
        pltpu.async_copy(table_hbm.at[idx_v.at[j]], row_v, gsem).wait()
        pltpu.sync_copy(row_v, out_hbm.at[wid, j])
        return carry

    lax.fori_loop(0, _NG, body, 0)


@functools.lru_cache(maxsize=1)
def _build_emb():
    return functools.partial(
        pl.kernel,
        mesh=plsc.VectorSubcoreMesh(core_axis_name="c", subcore_axis_name="s"),
        out_type=jax.ShapeDtypeStruct((_NW, _NG, _CHUNK, _EMB), jnp.float32),
        scratch_types=[
            pltpu.VMEM((_NG, _CHUNK), jnp.int32),
            pltpu.VMEM((_CHUNK, _EMB), jnp.float32),
            pltpu.SemaphoreType.DMA,
        ],
        compiler_params=pltpu.CompilerParams(use_tc_tiling_on_sc=False),
    )(_emb_body)


def kernel(input, hidden, table):
    del hidden
    idx = input.astype(jnp.int32).reshape(_NW, _NG, _CHUNK)
    out = _build_emb()(idx, table)
    return out.reshape(_BATCH, _SEQ, _EMB)


# speedup vs baseline: 1.1454x; 1.1454x over previous
import functools

import jax
import jax.numpy as jnp
from jax import lax
from jax.experimental import pallas as pl
from jax.experimental.pallas import tpu as pltpu
from jax.experimental.pallas import tpu_sc as plsc

_VOCAB = 1000000
_EMB = 16
_BATCH = 4096
_SEQ = 200
_B = _BATCH * _SEQ
_NC = 2
_NS = 16
_NW = _NC * _NS
_BPW = _B // _NW
_CHUNK = 128
_NG = _BPW // _CHUNK
_NBUF = 8
_AHEAD = 4


def _emb_body(idx_hbm, table_hbm, out_hbm, idx_v, row_v, gsems, ssems):
    wid = lax.axis_index("s") * _NC + lax.axis_index("c")
    pltpu.sync_copy(idx_hbm.at[wid], idx_v)

    def start_gather(k, b):
        pltpu.async_copy(table_hbm.at[idx_v.at[k]], row_v.at[b], gsems[b])

    def wait_gather(b):
        pltpu.make_async_copy(
            table_hbm.at[pl.ds(0, _CHUNK)], row_v.at[b], gsems[b]).wait()

    def start_store(j, b):
        pltpu.async_copy(row_v.at[b], out_hbm.at[wid, j], ssems[b])

    def wait_store(b):
        pltpu.make_async_copy(
            row_v.at[b], out_hbm.at[wid, 0], ssems[b]).wait()

    for b in range(_AHEAD):
        start_gather(b, b)

    for j in range(_NBUF - _AHEAD):
        s, bk = j % _NBUF, (j + _AHEAD) % _NBUF
        start_gather(j + _AHEAD, bk)
        wait_gather(s)
        start_store(j, s)

    def outer(g, carry):
        for t in range(_NBUF):
            j = (_NBUF - _AHEAD) + g * _NBUF + t
            s = (_NBUF - _AHEAD + t) % _NBUF
            bk = (s + _AHEAD) % _NBUF
            wait_store(bk)
            start_gather(j + _AHEAD, bk)
            wait_gather(s)
            start_store(j, s)
        return carry

    lax.fori_loop(0, (_NG - _NBUF) // _NBUF, outer, 0)

    for j in range(_NG - _AHEAD, _NG):
        s = j % _NBUF
        wait_gather(s)
        start_store(j, s)

    for j in range(_NG - _NBUF, _NG):
        wait_store(j % _NBUF)


@functools.lru_cache(maxsize=1)
def _build_emb():
    return functools.partial(
        pl.kernel,
        mesh=plsc.VectorSubcoreMesh(core_axis_name="c", subcore_axis_name="s"),
        out_type=jax.ShapeDtypeStruct((_NW, _NG, _CHUNK, _EMB), jnp.float32),
        scratch_types=[
            pltpu.VMEM((_NG, _CHUNK), jnp.int32),
            pltpu.VMEM((_NBUF, _CHUNK, _EMB), jnp.float32),
            [pltpu.SemaphoreType.DMA] * _NBUF,
            [pltpu.SemaphoreType.DMA] * _NBUF,
        ],
        compiler_params=pltpu.CompilerParams(use_tc_tiling_on_sc=False),
    )(_emb_body)


def kernel(input, hidden, table):
    del hidden
    idx = input.astype(jnp.int32).reshape(_NW, _NG, _CHUNK)
    out = _build_emb()(idx, table)
    return out.reshape(_BATCH, _SEQ, _EMB)


# speedup vs baseline: 1.5937x; 1.3914x over previous
import functools

import jax
import jax.numpy as jnp
from jax import lax
from jax.experimental import pallas as pl
from jax.experimental.pallas import tpu as pltpu
from jax.experimental.pallas import tpu_sc as plsc

_VOCAB = 1000000
_EMB = 16
_BATCH = 4096
_SEQ = 200
_NC = 2
_NS = 16
_NW = _NC * _NS
_CHUNK = 128
_NG = _SEQ
_NBUF = 4
_AHEAD = 2


def _emb_body(idx_hbm, table_hbm, out_hbm, idx_v, grow, tbuf, gsems, ssems):
    wid = lax.axis_index("s") * _NC + lax.axis_index("c")
    pltpu.sync_copy(idx_hbm.at[:, pl.ds(wid * _CHUNK, _CHUNK)], idx_v)

    lane = lax.iota(jnp.int32, 16)

    def start_gather(k, b):
        pltpu.async_copy(table_hbm.at[idx_v.at[k]], grow.at[b], gsems[b])

    def wait_gather(b):
        pltpu.make_async_copy(
            table_hbm.at[pl.ds(0, _CHUNK)], grow.at[b], gsems[b]).wait()

    def transpose(b):
        src = grow.at[b]
        for eh in range(2):
            for el in range(8):
                dst = tbuf.at[b, eh, el]
                col = jnp.full((16,), eh * 8 + el, jnp.int32)
                for b0 in range(0, _CHUNK, 16):
                    v = plsc.load_gather(src, [lane + b0, col])
                    dst[pl.ds(b0, 16)] = v

    def start_store(j, b):
        for eh in range(2):
            pltpu.async_copy(
                tbuf.at[b, eh], out_hbm.at[j, eh, wid], ssems[b])

    def wait_store(b):
        for eh in range(2):
            pltpu.make_async_copy(
                tbuf.at[b, eh], out_hbm.at[0, eh, wid], ssems[b]).wait()

    def step(j, s_, issue, swait):
        if issue:
            start_gather(j + _AHEAD, (s_ + _AHEAD) % _NBUF)
        wait_gather(s_)
        if swait:
            wait_store(s_)
        transpose(s_)
        start_store(j, s_)

    for b in range(_AHEAD):
        start_gather(b, b)

    for j in range(_NBUF):
        step(j, j % _NBUF, True, False)

    def outer(g, carry):
        for t in range(_NBUF):
            step(_NBUF + g * _NBUF + t, t, True, True)
        return carry

    lax.fori_loop(0, (_NG - 2 * _NBUF) // _NBUF, outer, 0)

    for j in range(_NG - _NBUF, _NG):
        step(j, j % _NBUF, j + _AHEAD < _NG, True)

    for j in range(_NG - _NBUF, _NG):
        wait_store(j % _NBUF)


@functools.lru_cache(maxsize=1)
def _build_emb():
    return functools.partial(
        pl.kernel,
        mesh=plsc.VectorSubcoreMesh(core_axis_name="c", subcore_axis_name="s"),
        out_type=jax.ShapeDtypeStruct((_SEQ, 2, _NW, 8, _CHUNK), jnp.float32),
        scratch_types=[
            pltpu.VMEM((_NG, _CHUNK), jnp.int32),
            pltpu.VMEM((_NBUF, _CHUNK, _EMB), jnp.float32),
            pltpu.VMEM((_NBUF, 2, 8, _CHUNK), jnp.float32),
            [pltpu.SemaphoreType.DMA] * _NBUF,
            [pltpu.SemaphoreType.DMA] * _NBUF,
        ],
        compiler_params=pltpu.CompilerParams(
            use_tc_tiling_on_sc=False, needs_layout_passes=False),
    )(_emb_body)


def kernel(input, hidden, table):
    del hidden
    idx = input.T.astype(jnp.int32)
    out5 = _build_emb()(idx, table)
    return out5.transpose(2, 4, 0, 1, 3).reshape(_BATCH, _SEQ, _EMB)


# speedup vs baseline: 1.5995x; 1.0036x over previous
import functools

import jax
import jax.numpy as jnp
from jax import lax
from jax.experimental import pallas as pl
from jax.experimental.pallas import tpu as pltpu
from jax.experimental.pallas import tpu_sc as plsc

_VOCAB = 1000000
_EMB = 16
_BATCH = 4096
_SEQ = 200
_NC = 2
_NS = 16
_NW = _NC * _NS
_CHUNK = 128
_NG = _SEQ
_NBUF = 4
_AHEAD = 3


def _emb_body(idx_hbm, table_hbm, out_hbm, idx_v, grow, tbuf, gsems, ssems):
    wid = lax.axis_index("s") * _NC + lax.axis_index("c")
    pltpu.sync_copy(idx_hbm.at[:, pl.ds(wid * _CHUNK, _CHUNK)], idx_v)

    lane = lax.iota(jnp.int32, 16)
    rowv = [lane + b0 for b0 in range(0, _CHUNK, 16)]
    colv = [jnp.full((16,), e, jnp.int32) for e in range(_EMB)]

    def start_gather(k, b):
        pltpu.async_copy(table_hbm.at[idx_v.at[k]], grow.at[b], gsems[b])

    def wait_gather(b):
        pltpu.make_async_copy(
            table_hbm.at[pl.ds(0, _CHUNK)], grow.at[b], gsems[b]).wait()

    def transpose(b):
        src = grow.at[b]
        for b0i in range(8):
            for e in range(_EMB):
                v = plsc.load_gather(src, [rowv[b0i], colv[e]])
                tbuf.at[b, e // 8, e % 8][pl.ds(b0i * 16, 16)] = v

    def start_store(j, b):
        pltpu.async_copy(tbuf.at[b], out_hbm.at[j, :, wid], ssems[b])

    def wait_store(b):
        pltpu.make_async_copy(
            tbuf.at[b], out_hbm.at[0, :, wid], ssems[b]).wait()

    def step(j, s_, issue, swait):
        if issue:
            start_gather(j + _AHEAD, (s_ + _AHEAD) % _NBUF)
        wait_gather(s_)
        if swait:
            wait_store(s_)
        transpose(s_)
        start_store(j, s_)

    for b in range(_AHEAD):
        start_gather(b, b)

    for j in range(_NBUF):
        step(j, j % _NBUF, True, False)

    def outer(g, carry):
        for t in range(_NBUF):
            step(_NBUF + g * _NBUF + t, t, True, True)
        return carry

    lax.fori_loop(0, (_NG - 2 * _NBUF) // _NBUF, outer, 0)

    for j in range(_NG - _NBUF, _NG):
        step(j, j % _NBUF, j + _AHEAD < _NG, True)

    for j in range(_NG - _NBUF, _NG):
        wait_store(j % _NBUF)


@functools.lru_cache(maxsize=1)
def _build_emb():
    return functools.partial(
        pl.kernel,
        mesh=plsc.VectorSubcoreMesh(core_axis_name="c", subcore_axis_name="s"),
        out_type=jax.ShapeDtypeStruct((_SEQ, 2, _NW, 8, _CHUNK), jnp.float32),
        scratch_types=[
            pltpu.VMEM((_NG, _CHUNK), jnp.int32),
            pltpu.VMEM((_NBUF, _CHUNK, _EMB), jnp.float32),
            pltpu.VMEM((_NBUF, 2, 8, _CHUNK), jnp.float32),
            [pltpu.SemaphoreType.DMA] * _NBUF,
            [pltpu.SemaphoreType.DMA] * _NBUF,
        ],
        compiler_params=pltpu.CompilerParams(
            use_tc_tiling_on_sc=False, needs_layout_passes=False),
    )(_emb_body)


def kernel(input, hidden, table):
    del hidden
    idx = input.T.astype(jnp.int32)
    out5 = _build_emb()(idx, table)
    return out5.transpose(2, 4, 0, 1, 3).reshape(_BATCH, _SEQ, _EMB)


# speedup vs baseline: 1.9140x; 1.1966x over previous
import functools

import jax
import jax.numpy as jnp
from jax import lax
from jax.experimental import pallas as pl
from jax.experimental.pallas import tpu as pltpu
from jax.experimental.pallas import tpu_sc as plsc

_VOCAB = 1000000
_EMB = 16
_BATCH = 4096
_SEQ = 200
_NC = 2
_NS = 16
_NW = _NC * _NS
_CHUNK = 128
_NG = _SEQ
_NBUF = 4
_AHEAD = 3


def _emb_body(idx_hbm, table_hbm, out_hbm, idx_v, grow, tbuf, gsems, ssems):
    wid = lax.axis_index("s") * _NC + lax.axis_index("c")
    pltpu.sync_copy(idx_hbm.at[:, pl.ds(wid * _CHUNK, _CHUNK)], idx_v)

    lane = lax.iota(jnp.int32, 16)
    ehv = lax.shift_right_logical(lane, 3)
    elv = lax.bitwise_and(lane, 7)

    def start_gather(k, b):
        pltpu.async_copy(table_hbm.at[idx_v.at[k]], grow.at[b], gsems[b])

    def wait_gather(b):
        pltpu.make_async_copy(
            table_hbm.at[pl.ds(0, _CHUNK)], grow.at[b], gsems[b]).wait()

    def transpose(b):
        dst = tbuf.at[b]
        for bi in range(_CHUNK):
            v = grow[b, bi, :]
            plsc.store_scatter(dst, [ehv, elv, jnp.full((16,), bi, jnp.int32)], v)

    def start_store(j, b):
        pltpu.async_copy(
            tbuf.at[b, :, :, pl.ds(0, _CHUNK)], out_hbm.at[j, :, wid], ssems[b])

    def wait_store(b):
        pltpu.make_async_copy(
            tbuf.at[b, :, :, pl.ds(0, _CHUNK)], out_hbm.at[0, :, wid],
            ssems[b]).wait()

    def step(j, s_, issue, swait):
        if issue:
            start_gather(j + _AHEAD, (s_ + _AHEAD) % _NBUF)
        wait_gather(s_)
        if swait:
            wait_store(s_)
        transpose(s_)
        start_store(j, s_)

    for b in range(_AHEAD):
        start_gather(b, b)

    for j in range(_NBUF):
        step(j, j % _NBUF, True, False)

    def outer(g, carry):
        for t in range(_NBUF):
            step(_NBUF + g * _NBUF + t, t, True, True)
        return carry

    lax.fori_loop(0, (_NG - 2 * _NBUF) // _NBUF, outer, 0)

    for j in range(_NG - _NBUF, _NG):
        step(j, j % _NBUF, j + _AHEAD < _NG, True)

    for j in range(_NG - _NBUF, _NG):
        wait_store(j % _NBUF)


@functools.lru_cache(maxsize=1)
def _build_emb():
    return functools.partial(
        pl.kernel,
        mesh=plsc.VectorSubcoreMesh(core_axis_name="c", subcore_axis_name="s"),
        out_type=jax.ShapeDtypeStruct((_SEQ, 2, _NW, 8, _CHUNK), jnp.float32),
        scratch_types=[
            pltpu.VMEM((_NG, _CHUNK), jnp.int32),
            pltpu.VMEM((_NBUF, _CHUNK, _EMB), jnp.float32),
            pltpu.VMEM((_NBUF, 2, 8, _CHUNK + 1), jnp.float32),
            [pltpu.SemaphoreType.DMA] * _NBUF,
            [pltpu.SemaphoreType.DMA] * _NBUF,
        ],
        compiler_params=pltpu.CompilerParams(
            use_tc_tiling_on_sc=False, needs_layout_passes=False),
    )(_emb_body)


def kernel(input, hidden, table):
    del hidden
    idx = input.T.astype(jnp.int32)
    out5 = _build_emb()(idx, table)
    return out5.transpose(2, 4, 0, 1, 3).reshape(_BATCH, _SEQ, _EMB)


# speedup vs baseline: 2.5207x; 1.3170x over previous
import functools

import jax
import jax.numpy as jnp
from jax import lax
from jax.experimental import pallas as pl
from jax.experimental.pallas import tpu as pltpu
from jax.experimental.pallas import tpu_sc as plsc

_VOCAB = 1000000
_EMB = 16
_BATCH = 4096
_SEQ = 200
_NC = 2
_NS = 16
_NW = _NC * _NS
_CHUNK = 128
_NG = _SEQ
_NBUF = 4
_AHEAD = 3


def _emb_body(idx_hbm, table_hbm, tail_hbm, out_hbm, idx_v, grow, tbuf, tail_v,
              gsems, ssems):
    wid = lax.axis_index("s") * _NC + lax.axis_index("c")
    pltpu.sync_copy(idx_hbm.at[:, pl.ds(wid * _CHUNK, _CHUNK)], idx_v)
    pltpu.sync_copy(tail_hbm, tail_v)

    lane = lax.iota(jnp.int32, 16)
    ehv = lax.shift_right_logical(lane, 3)
    elv = lax.bitwise_and(lane, 7)

    def start_gather(k, b):
        pltpu.async_copy(table_hbm.at[idx_v.at[k]], grow.at[b], gsems[b])

    def wait_gather(b):
        pltpu.make_async_copy(
            table_hbm.at[pl.ds(0, _CHUNK)], grow.at[b], gsems[b]).wait()

    def transpose(b):
        dst = tbuf.at[b]
        for bi in range(_CHUNK):
            v = grow[b, bi, :]
            plsc.store_scatter(dst, [ehv, elv, jnp.full((16,), bi, jnp.int32)], v)

    def start_store(j, b):
        pltpu.async_copy(
            tbuf.at[b, :, :, pl.ds(0, _CHUNK)], out_hbm.at[j, :, wid], ssems[b])

    def wait_store(b):
        pltpu.make_async_copy(
            tbuf.at[b, :, :, pl.ds(0, _CHUNK)], out_hbm.at[0, :, wid],
            ssems[b]).wait()

    def fixup(j, b):
        iv = [idx_v[j, pl.ds(tt * 16, 16)] for tt in range(8)]
        m = [v >= _TAIL0 for v in iv]
        any_m = m[0]
        for tt in range(1, 8):
            any_m = jnp.logical_or(any_m, m[tt])
        cnt = jnp.sum(any_m.astype(jnp.int32))

        @pl.when(cnt > 0)
        def _():
            dst = tbuf.at[b]

            def fi(tt, c):
                ivd = idx_v[j, pl.ds(tt * 16, 16)]
                md = ivd >= _TAIL0
                rloc = ivd - _TAIL0
                for e in range(_EMB):
                    v = plsc.load_gather(
                        tail_v, [rloc, jnp.full((16,), e, jnp.int32)],
                        mask=md)
                    plsc.store_scatter(
                        dst,
                        [jnp.full((16,), e // 8, jnp.int32),
                         jnp.full((16,), e % 8, jnp.int32),
                         lane + tt * 16],
                        v, mask=md)
                return c

            lax.fori_loop(0, 8, fi, 0)

    def step(j, s_, issue, swait):
        if issue:
            start_gather(j + _AHEAD, (s_ + _AHEAD) % _NBUF)
        wait_gather(s_)
        if swait:
            wait_store(s_)
        transpose(s_)
        fixup(j, s_)
        start_store(j, s_)

    for b in range(_AHEAD):
        start_gather(b, b)

    for j in range(_NBUF):
        step(j, j % _NBUF, True, False)

    def outer(g, carry):
        for t in range(_NBUF):
            step(_NBUF + g * _NBUF + t, t, True, True)
        return carry

    lax.fori_loop(0, (_NG - 2 * _NBUF) // _NBUF, outer, 0)

    for j in range(_NG - _NBUF, _NG):
        step(j, j % _NBUF, j + _AHEAD < _NG, True)

    for j in range(_NG - _NBUF, _NG):
        wait_store(j % _NBUF)


_FCOLS = 7812
_TAIL0 = _FCOLS * _CHUNK
_FPW = 248
_FNBUF = 4
_FAHEAD = 3


def _fmt_body(tt_hbm, out_hbm, in_bufs, tr_bufs, isems, osems):
    wid = lax.axis_index("s") * _NC + lax.axis_index("c")
    lane = lax.iota(jnp.int32, 16)
    perm = [lax.bitwise_and(lane + d, 15) for d in range(16)]
    sbase = [perm[d] * 16 + lane for d in range(16)]

    def rb(k):
        return jnp.minimum(wid * _FPW + k, _FCOLS - 1) * _CHUNK

    def start_in(k, b):
        pltpu.async_copy(
            tt_hbm.at[:, pl.ds(rb(k), _CHUNK)], in_bufs[b], isems[b])

    def wait_in(b):
        pltpu.make_async_copy(
            tt_hbm.at[:, pl.ds(0, _CHUNK)], in_bufs[b], isems[b]).wait()

    def transpose(b):
        src = in_bufs[b]
        dst = tr_bufs[b]
        for c0 in range(0, _CHUNK, 16):
            for d in range(16):
                v = plsc.load_gather(src, [lane, perm[d] + c0])
                plsc.store_scatter(dst, [sbase[d] + c0 * 16], v)

    def start_out(k, b):
        pltpu.async_copy(
            tr_bufs[b], out_hbm.at[pl.ds(rb(k) * _EMB, _CHUNK * _EMB)],
            osems[b])

    def wait_out(b):
        pltpu.make_async_copy(
            tr_bufs[b], out_hbm.at[pl.ds(0, _CHUNK * _EMB)], osems[b]).wait()

    def step(k, s_, issue, owait):
        if issue:
            start_in(k + _FAHEAD, (s_ + _FAHEAD) % _FNBUF)
        wait_in(s_)
        if owait:
            wait_out(s_)
        transpose(s_)
        start_out(k, s_)

    for b in range(_FAHEAD):
        start_in(b, b)
    for k in range(_FNBUF):
        step(k, k % _FNBUF, True, False)

    def outer(g, carry):
        for t in range(_FNBUF):
            step(_FNBUF + g * _FNBUF + t, t, True, True)
        return carry

    lax.fori_loop(0, (_FPW - 2 * _FNBUF) // _FNBUF, outer, 0)

    for k in range(_FPW - _FNBUF, _FPW):
        step(k, k % _FNBUF, k + _FAHEAD < _FPW, True)
    for k in range(_FPW - _FNBUF, _FPW):
        wait_out(k % _FNBUF)


@functools.lru_cache(maxsize=1)
def _build_fmt():
    return functools.partial(
        pl.kernel,
        mesh=plsc.VectorSubcoreMesh(core_axis_name="c", subcore_axis_name="s"),
        out_type=jax.ShapeDtypeStruct((_VOCAB * _EMB,), jnp.float32),
        scratch_types=[
            [pltpu.VMEM((_EMB, _CHUNK), jnp.float32)] * _FNBUF,
            [pltpu.VMEM((_CHUNK * _EMB,), jnp.float32)] * _FNBUF,
            [pltpu.SemaphoreType.DMA] * _FNBUF,
            [pltpu.SemaphoreType.DMA] * _FNBUF,
        ],
        compiler_params=pltpu.CompilerParams(
            use_tc_tiling_on_sc=True, needs_layout_passes=False),
    )(_fmt_body)


@functools.lru_cache(maxsize=1)
def _build_emb():
    return functools.partial(
        pl.kernel,
        mesh=plsc.VectorSubcoreMesh(core_axis_name="c", subcore_axis_name="s"),
        out_type=jax.ShapeDtypeStruct((_SEQ, 2, _NW, 8, _CHUNK), jnp.float32),
        scratch_types=[
            pltpu.VMEM((_NG, _CHUNK), jnp.int32),
            pltpu.VMEM((_NBUF, _CHUNK, _EMB), jnp.float32),
            pltpu.VMEM((_NBUF, 2, 8, _CHUNK + 1), jnp.float32),
            pltpu.VMEM((64, _EMB), jnp.float32),
            [pltpu.SemaphoreType.DMA] * _NBUF,
            [pltpu.SemaphoreType.DMA] * _NBUF,
        ],
        compiler_params=pltpu.CompilerParams(
            use_tc_tiling_on_sc=False, needs_layout_passes=False),
    )(_emb_body)


def kernel(input, hidden, table):
    del hidden
    idx = input.T.astype(jnp.int32)
    table_rm = _build_fmt()(table.T).reshape(_VOCAB, _EMB)
    tail = table[_TAIL0:]
    out5 = _build_emb()(idx, table_rm, tail)
    return out5.transpose(2, 4, 0, 1, 3).reshape(_BATCH, _SEQ, _EMB)


# speedup vs baseline: 2.8334x; 1.1241x over previous
import functools

import jax
import jax.numpy as jnp
from jax import lax
from jax.experimental import pallas as pl
from jax.experimental.pallas import tpu as pltpu
from jax.experimental.pallas import tpu_sc as plsc

_VOCAB = 1000000
_EMB = 16
_BATCH = 4096
_SEQ = 200
_NC = 2
_NS = 16
_NW = _NC * _NS
_CHUNK = 128
_NG = _SEQ
_NBUF = 4
_AHEAD = 3


def _emb_body(idx_hbm, table_hbm, out_hbm, idx_v, grow, tbuf, gsems, ssems):
    wid = lax.axis_index("s") * _NC + lax.axis_index("c")
    pltpu.sync_copy(idx_hbm.at[:, pl.ds(wid * _CHUNK, _CHUNK)], idx_v)

    lane = lax.iota(jnp.int32, 16)
    ehv = lax.shift_right_logical(lane, 3)
    elv = lax.bitwise_and(lane, 7)

    def start_gather(k, b):
        pltpu.async_copy(table_hbm.at[idx_v.at[k]], grow.at[b], gsems[b])

    def wait_gather(b):
        pltpu.make_async_copy(
            table_hbm.at[pl.ds(0, _CHUNK)], grow.at[b], gsems[b]).wait()

    def transpose(b):
        dst = tbuf.at[b]
        for bi in range(_CHUNK):
            v = grow[b, bi, :]
            plsc.store_scatter(dst, [ehv, elv, jnp.full((16,), bi, jnp.int32)], v)

    def start_store(j, b):
        pltpu.async_copy(
            tbuf.at[b, :, :, pl.ds(0, _CHUNK)], out_hbm.at[j, :, wid], ssems[b])

    def wait_store(b):
        pltpu.make_async_copy(
            tbuf.at[b, :, :, pl.ds(0, _CHUNK)], out_hbm.at[0, :, wid],
            ssems[b]).wait()

    def step(j, s_, issue, swait):
        if issue:
            start_gather(j + _AHEAD, (s_ + _AHEAD) % _NBUF)
        wait_gather(s_)
        if swait:
            wait_store(s_)
        transpose(s_)
        start_store(j, s_)

    for b in range(_AHEAD):
        start_gather(b, b)

    for j in range(_NBUF):
        step(j, j % _NBUF, True, False)

    def outer(g, carry):
        for t in range(_NBUF):
            step(_NBUF + g * _NBUF + t, t, True, True)
        return carry

    lax.fori_loop(0, (_NG - 2 * _NBUF) // _NBUF, outer, 0)

    for j in range(_NG - _NBUF, _NG):
        step(j, j % _NBUF, j + _AHEAD < _NG, True)

    for j in range(_NG - _NBUF, _NG):
        wait_store(j % _NBUF)


_FCOLS = 7812
_TAIL0 = _FCOLS * _CHUNK
_FPW = 248
_FNBUF = 4
_FAHEAD = 3


def _fmt_body(tt_hbm, tail_hbm, out_hbm, in_bufs, tr_bufs, isems, osems):
    wid = lax.axis_index("s") * _NC + lax.axis_index("c")
    lane = lax.iota(jnp.int32, 16)
    perm = [lax.bitwise_and(lane + d, 15) for d in range(16)]
    sbase = [perm[d] * 16 + lane for d in range(16)]

    def rb(k):
        return jnp.minimum(wid * _FPW + k, _FCOLS - 1) * _CHUNK

    def start_in(k, b):
        pltpu.async_copy(
            tt_hbm.at[:, pl.ds(rb(k), _CHUNK)], in_bufs[b], isems[b])

    def wait_in(b):
        pltpu.make_async_copy(
            tt_hbm.at[:, pl.ds(0, _CHUNK)], in_bufs[b], isems[b]).wait()

    def transpose(b):
        src = in_bufs[b]
        dst = tr_bufs[b]
        for c0 in range(0, _CHUNK, 16):
            for d in range(16):
                v = plsc.load_gather(src, [lane, perm[d] + c0])
                plsc.store_scatter(dst, [sbase[d] + c0 * 16], v)

    def start_out(k, b):
        pltpu.async_copy(
            tr_bufs[b], out_hbm.at[pl.ds(rb(k) * _EMB, _CHUNK * _EMB)],
            osems[b])

    def wait_out(b):
        pltpu.make_async_copy(
            tr_bufs[b], out_hbm.at[pl.ds(0, _CHUNK * _EMB)], osems[b]).wait()

    def step(k, s_, issue, owait):
        if issue:
            start_in(k + _FAHEAD, (s_ + _FAHEAD) % _FNBUF)
        wait_in(s_)
        if owait:
            wait_out(s_)
        transpose(s_)
        start_out(k, s_)

    for b in range(_FAHEAD):
        start_in(b, b)
    for k in range(_FNBUF):
        step(k, k % _FNBUF, True, False)

    def outer(g, carry):
        for t in range(_FNBUF):
            step(_FNBUF + g * _FNBUF + t, t, True, True)
        return carry

    lax.fori_loop(0, (_FPW - 2 * _FNBUF) // _FNBUF, outer, 0)

    for k in range(_FPW - _FNBUF, _FPW):
        step(k, k % _FNBUF, k + _FAHEAD < _FPW, True)
    for k in range(_FPW - _FNBUF, _FPW):
        wait_out(k % _FNBUF)

    pltpu.sync_copy(tail_hbm, tr_bufs[0].at[pl.ds(0, 64 * _EMB)])
    pltpu.sync_copy(tr_bufs[0].at[pl.ds(0, 64 * _EMB)],
                    out_hbm.at[pl.ds(_TAIL0 * _EMB, 64 * _EMB)])


@functools.lru_cache(maxsize=1)
def _build_fmt():
    return functools.partial(
        pl.kernel,
        mesh=plsc.VectorSubcoreMesh(core_axis_name="c", subcore_axis_name="s"),
        out_type=jax.ShapeDtypeStruct((_VOCAB * _EMB,), jnp.float32),
        scratch_types=[
            [pltpu.VMEM((_EMB, _CHUNK), jnp.float32)] * _FNBUF,
            [pltpu.VMEM((_CHUNK * _EMB,), jnp.float32)] * _FNBUF,
            [pltpu.SemaphoreType.DMA] * _FNBUF,
            [pltpu.SemaphoreType.DMA] * _FNBUF,
        ],
        compiler_params=pltpu.CompilerParams(
            use_tc_tiling_on_sc=True, needs_layout_passes=False,
            disable_bounds_checks=True),
    )(_fmt_body)


@functools.lru_cache(maxsize=1)
def _build_emb():
    return functools.partial(
        pl.kernel,
        mesh=plsc.VectorSubcoreMesh(core_axis_name="c", subcore_axis_name="s"),
        out_type=jax.ShapeDtypeStruct((_SEQ, 2, _NW, 8, _CHUNK), jnp.float32),
        scratch_types=[
            pltpu.VMEM((_NG, _CHUNK), jnp.int32),
            pltpu.VMEM((_NBUF, _CHUNK, _EMB), jnp.float32),
            pltpu.VMEM((_NBUF, 2, 8, _CHUNK + 1), jnp.float32),
            [pltpu.SemaphoreType.DMA] * _NBUF,
            [pltpu.SemaphoreType.DMA] * _NBUF,
        ],
        compiler_params=pltpu.CompilerParams(
            use_tc_tiling_on_sc=False, needs_layout_passes=False,
            disable_bounds_checks=True),
    )(_emb_body)


def kernel(input, hidden, table):
    del hidden
    idx = input.T.astype(jnp.int32)
    tail = table[_TAIL0:].reshape(64 * _EMB)
    table_rm = _build_fmt()(table.T, tail).reshape(_VOCAB, _EMB)
    out5 = _build_emb()(idx, table_rm)
    return out5.transpose(2, 4, 0, 1, 3).reshape(_BATCH, _SEQ, _EMB)
